# Initial kernel scaffold; baseline (speedup 1.0000x reference)
import jax
import jax.numpy as jnp
from jax.experimental import pallas as pl


def kernel(x, embed_weight):
    raise NotImplementedError("write your pallas kernel here")


# speedup vs baseline: 1.6862x; 1.6862x over previous
import jax
import jax.numpy as jnp
from jax.experimental import pallas as pl


_BS = 512


def _add_kernel(x_ref, e_ref, o_ref):
    o_ref[...] = x_ref[...] + e_ref[...]


def kernel(x, embed_weight):
    B, S, D = x.shape
    grid = (S // _BS, B)
    return pl.pallas_call(
        _add_kernel,
        grid=grid,
        in_specs=[
            pl.BlockSpec((1, _BS, D), lambda s, b: (b, s, 0)),
            pl.BlockSpec((_BS, D), lambda s, b: (s, 0)),
        ],
        out_specs=pl.BlockSpec((1, _BS, D), lambda s, b: (b, s, 0)),
        out_shape=jax.ShapeDtypeStruct((B, S, D), x.dtype),
    )(x, embed_weight)


# speedup vs baseline: 1.8757x; 1.1124x over previous
import jax
import jax.numpy as jnp
from jax.experimental import pallas as pl


_BS = 1024


def _add_kernel(x_ref, e_ref, o_ref):
    o_ref[...] = x_ref[...] + e_ref[...]


def kernel(x, embed_weight):
    B, S, D = x.shape
    grid = (S // _BS, B)
    return pl.pallas_call(
        _add_kernel,
        grid=grid,
        in_specs=[
            pl.BlockSpec((1, _BS, D), lambda s, b: (b, s, 0)),
            pl.BlockSpec((_BS, D), lambda s, b: (s, 0)),
        ],
        out_specs=pl.BlockSpec((1, _BS, D), lambda s, b: (b, s, 0)),
        out_shape=jax.ShapeDtypeStruct((B, S, D), x.dtype),
    )(x, embed_weight)


# speedup vs baseline: 1.9932x; 1.0627x over previous
import jax
import jax.numpy as jnp
from jax.experimental import pallas as pl


_BS = 2048


def _add_kernel(x_ref, e_ref, o_ref):
    o_ref[...] = x_ref[...] + e_ref[...]


def kernel(x, embed_weight):
    B, S, D = x.shape
    grid = (S // _BS, B)
    return pl.pallas_call(
        _add_kernel,
        grid=grid,
        in_specs=[
            pl.BlockSpec((1, _BS, D), lambda s, b: (b, s, 0)),
            pl.BlockSpec((_BS, D), lambda s, b: (s, 0)),
        ],
        out_specs=pl.BlockSpec((1, _BS, D), lambda s, b: (b, s, 0)),
        out_shape=jax.ShapeDtypeStruct((B, S, D), x.dtype),
    )(x, embed_weight)
